# baseline (speedup 1.0000x reference)
import jax
import jax.numpy as jnp
from jax.experimental import pallas as pl
from jax.experimental.pallas import tpu as pltpu

_MARGIN = 1.0
_BLOCK = 256


def _loss_block_kernel(x_blk_ref, x_ref, lab_row_ref, lab_col_ref, out_ref):
    i = pl.program_id(0)
    xb = x_blk_ref[...]
    x = x_ref[...]
    blk, n = xb.shape[0], x.shape[0]

    g = jax.lax.dot_general(
        xb, x, (((1,), (1,)), ((), ())), preferred_element_type=jnp.float32
    )
    sq_row = jnp.sum(xb * xb, axis=1)[:, None]
    sq_col = jnp.sum(x * x, axis=1)[None, :]
    d2 = jnp.maximum(sq_row + sq_col - 2.0 * g, 0.0)

    lab_row = lab_row_ref[...]
    lab_col = lab_col_ref[...]
    row_ids = i * blk + jax.lax.broadcasted_iota(jnp.int32, (blk, n), 0)
    col_ids = jax.lax.broadcasted_iota(jnp.int32, (blk, n), 1)
    eye = row_ids == col_ids

    same = (lab_row == lab_col) & ~eye
    diff = lab_row != lab_col

    pos = jnp.where(same, d2, 0.0)
    d = jnp.sqrt(d2)
    hinge = jnp.maximum(_MARGIN - d, 0.0)
    neg = jnp.where(diff, hinge * hinge, 0.0)

    total = jnp.sum(pos + neg, axis=0, keepdims=True)
    total = jnp.sum(total.reshape(1, n // 128, 128), axis=1)
    out_ref[...] = total[None]


def kernel(output, label):
    n, d = output.shape
    lab = jnp.asarray(label, jnp.int32)
    lab_row = lab.reshape(n, 1)
    lab_col = lab.reshape(1, n)
    g = n // _BLOCK

    partials = pl.pallas_call(
        _loss_block_kernel,
        grid=(g,),
        in_specs=[
            pl.BlockSpec((_BLOCK, d), lambda i: (i, 0)),
            pl.BlockSpec((n, d), lambda i: (0, 0)),
            pl.BlockSpec((_BLOCK, 1), lambda i: (i, 0)),
            pl.BlockSpec((1, n), lambda i: (0, 0)),
        ],
        out_specs=pl.BlockSpec((1, 1, 128), lambda i: (i, 0, 0)),
        out_shape=jax.ShapeDtypeStruct((g, 1, 128), jnp.float32),
        compiler_params=pltpu.CompilerParams(
            dimension_semantics=("parallel",)
        ),
    )(output, output, lab_row, lab_col)

    return jnp.sum(partials) / (n * (n - 1))


# speedup vs baseline: 2.0473x; 2.0473x over previous
import jax
import jax.numpy as jnp
from jax.experimental import pallas as pl
from jax.experimental.pallas import tpu as pltpu

_MARGIN = 1.0
_BLOCK = 256
_KPAD = 128


def _loss_block_kernel(a_blk_ref, b_ref, lab_row_ref, lab_col_ref, out_ref):
    ab = a_blk_ref[...]
    b = b_ref[...]
    n = b.shape[0]

    d2 = jax.lax.dot_general(
        ab, b, (((1,), (1,)), ((), ())), preferred_element_type=jnp.float32
    )
    d2 = jnp.maximum(d2, 0.0)

    d = jnp.sqrt(d2)
    h = jnp.maximum(_MARGIN - d, 0.0)
    eq = lab_row_ref[...] == lab_col_ref[...]
    v = jnp.where(eq, d2, h * h)

    total = jnp.sum(v, axis=0, keepdims=True)
    total = jnp.sum(total.reshape(1, n // 128, 128), axis=1)
    out_ref[...] = total[None]


def kernel(output, label):
    n, dim = output.shape
    x = jnp.asarray(output, jnp.float32)
    sq = jnp.sum(x * x, axis=1, keepdims=True)
    ones = jnp.ones((n, 1), jnp.float32)
    zpad = jnp.zeros((n, _KPAD - dim - 2), jnp.float32)
    a = jnp.concatenate([-2.0 * x, sq, ones, zpad], axis=1)
    b = jnp.concatenate([x, ones, sq, zpad], axis=1)

    lab = jnp.asarray(label, jnp.int32)
    lab_row = lab.reshape(n, 1)
    lab_col = lab.reshape(1, n)
    g = n // _BLOCK

    partials = pl.pallas_call(
        _loss_block_kernel,
        grid=(g,),
        in_specs=[
            pl.BlockSpec((_BLOCK, _KPAD), lambda i: (i, 0)),
            pl.BlockSpec((n, _KPAD), lambda i: (0, 0)),
            pl.BlockSpec((_BLOCK, 1), lambda i: (i, 0)),
            pl.BlockSpec((1, n), lambda i: (0, 0)),
        ],
        out_specs=pl.BlockSpec((1, 1, 128), lambda i: (i, 0, 0)),
        out_shape=jax.ShapeDtypeStruct((g, 1, 128), jnp.float32),
        compiler_params=pltpu.CompilerParams(
            dimension_semantics=("parallel",)
        ),
    )(a, b, lab_row, lab_col)

    return jnp.sum(partials) / (n * (n - 1))


# speedup vs baseline: 2.5989x; 1.2694x over previous
import jax
import jax.numpy as jnp
from jax.experimental import pallas as pl
from jax.experimental.pallas import tpu as pltpu

_MARGIN = 1.0
_BLK = 512
_KPAD = 128


def _tile_ij(g, p, q):
    cond = q < g - p
    i = jnp.where(cond, p, g - 1 - p)
    j = jnp.where(cond, p + q, q - 1)
    return i, j


def _loss_tile_kernel(a_blk_ref, b_blk_ref, lab_row_ref, lab_col_ref,
                      out_ref, *, g):
    p = pl.program_id(0)
    q = pl.program_id(1)
    ab = a_blk_ref[...]
    bb = b_blk_ref[...]

    d2 = jax.lax.dot_general(
        ab, bb, (((1,), (1,)), ((), ())), preferred_element_type=jnp.float32
    )
    d2 = jnp.maximum(d2, 0.0)

    d = jnp.sqrt(d2)
    h = jnp.maximum(_MARGIN - d, 0.0)
    eq = lab_row_ref[...] == lab_col_ref[...]
    v = jnp.where(eq, d2, h * h)

    blk = v.shape[1]
    total = jnp.sum(v, axis=0, keepdims=True)
    total = jnp.sum(total.reshape(1, blk // 128, 128), axis=1)

    i, j = _tile_ij(g, p, q)
    w = jnp.where(i == j, 1.0, 2.0).astype(jnp.float32)
    contrib = (w * total)[None]

    @pl.when(q == 0)
    def _init():
        out_ref[...] = contrib

    @pl.when(q != 0)
    def _acc():
        out_ref[...] += contrib


def kernel(output, label):
    n, dim = output.shape
    x = jnp.asarray(output, jnp.float32)
    sq = jnp.sum(x * x, axis=1, keepdims=True)
    ones = jnp.ones((n, 1), jnp.float32)
    zpad = jnp.zeros((n, _KPAD - dim - 2), jnp.float32)
    a = jnp.concatenate([-2.0 * x, sq, ones, zpad], axis=1)
    b = jnp.concatenate([x, ones, sq, zpad], axis=1)

    lab = jnp.asarray(label, jnp.int32)
    lab_row = lab.reshape(n, 1)
    lab_col = lab.reshape(1, n)
    g = n // _BLK

    import functools
    body = functools.partial(_loss_tile_kernel, g=g)

    partials = pl.pallas_call(
        body,
        grid=(g // 2, g + 1),
        in_specs=[
            pl.BlockSpec((_BLK, _KPAD), lambda p, q: (_tile_ij(g, p, q)[0], 0)),
            pl.BlockSpec((_BLK, _KPAD), lambda p, q: (_tile_ij(g, p, q)[1], 0)),
            pl.BlockSpec((_BLK, 1), lambda p, q: (_tile_ij(g, p, q)[0], 0)),
            pl.BlockSpec((1, _BLK), lambda p, q: (0, _tile_ij(g, p, q)[1])),
        ],
        out_specs=pl.BlockSpec((1, 1, 128), lambda p, q: (p, 0, 0)),
        out_shape=jax.ShapeDtypeStruct((g // 2, 1, 128), jnp.float32),
        compiler_params=pltpu.CompilerParams(
            dimension_semantics=("parallel", "arbitrary")
        ),
    )(a, b, lab_row, lab_col)

    return jnp.sum(partials) / (n * (n - 1))


# speedup vs baseline: 3.5410x; 1.3625x over previous
import jax
import jax.numpy as jnp
from jax.experimental import pallas as pl
from jax.experimental.pallas import tpu as pltpu

_MARGIN = 1.0
_BLK = 1024
_KPAD = 128


def _tile_ij(g, p, q):
    cond = q < g - p
    i = jnp.where(cond, p, g - 1 - p)
    j = jnp.where(cond, p + q, q - 1)
    return i, j


def _loss_tile_kernel(a_blk_ref, b_blk_ref, lab_row_ref, lab_col_ref,
                      out_ref, *, g):
    p = pl.program_id(0)
    q = pl.program_id(1)
    ab = a_blk_ref[...]
    bb = b_blk_ref[...]

    d2 = jax.lax.dot_general(
        ab, bb, (((1,), (1,)), ((), ())), preferred_element_type=jnp.float32
    )
    d2 = jnp.maximum(d2, 0.0)

    d = jnp.sqrt(d2)
    h = jnp.maximum(_MARGIN - d, 0.0)
    eq = lab_row_ref[...] == lab_col_ref[...]
    v = jnp.where(eq, d2, h * h)

    blk = v.shape[1]
    total = jnp.sum(v, axis=0, keepdims=True)
    total = jnp.sum(total.reshape(1, blk // 128, 128), axis=1)

    i, j = _tile_ij(g, p, q)
    w = jnp.where(i == j, 1.0, 2.0).astype(jnp.float32)
    contrib = (w * total)[None]

    @pl.when(q == 0)
    def _init():
        out_ref[...] = contrib

    @pl.when(q != 0)
    def _acc():
        out_ref[...] += contrib


def kernel(output, label):
    n, dim = output.shape
    x = jnp.asarray(output, jnp.float32)
    sq = jnp.sum(x * x, axis=1, keepdims=True)
    ones = jnp.ones((n, 1), jnp.float32)
    zpad = jnp.zeros((n, _KPAD - dim - 2), jnp.float32)
    a = jnp.concatenate([-2.0 * x, sq, ones, zpad], axis=1)
    b = jnp.concatenate([x, ones, sq, zpad], axis=1)

    lab = jnp.asarray(label, jnp.int32)
    lab_row = lab.reshape(n, 1)
    lab_col = lab.reshape(1, n)
    g = n // _BLK

    import functools
    body = functools.partial(_loss_tile_kernel, g=g)

    partials = pl.pallas_call(
        body,
        grid=(g // 2, g + 1),
        in_specs=[
            pl.BlockSpec((_BLK, _KPAD), lambda p, q: (_tile_ij(g, p, q)[0], 0)),
            pl.BlockSpec((_BLK, _KPAD), lambda p, q: (_tile_ij(g, p, q)[1], 0)),
            pl.BlockSpec((_BLK, 1), lambda p, q: (_tile_ij(g, p, q)[0], 0)),
            pl.BlockSpec((1, _BLK), lambda p, q: (0, _tile_ij(g, p, q)[1])),
        ],
        out_specs=pl.BlockSpec((1, 1, 128), lambda p, q: (p, 0, 0)),
        out_shape=jax.ShapeDtypeStruct((g // 2, 1, 128), jnp.float32),
        compiler_params=pltpu.CompilerParams(
            dimension_semantics=("parallel", "arbitrary")
        ),
    )(a, b, lab_row, lab_col)

    return jnp.sum(partials) / (n * (n - 1))
